# Initial kernel scaffold; baseline (speedup 1.0000x reference)
import jax
import jax.numpy as jnp
from jax.experimental import pallas as pl


def kernel(x, edge_index, batch, W1_0, b1_0, W2_0, b2_0, gamma_0, beta_0, W1_1, b1_1, W2_1, b2_1, gamma_1, beta_1, W1_2, b1_2, W2_2, b2_2, gamma_2, beta_2, W_mu, b_mu, W_lv, b_lv):
    raise NotImplementedError("write your pallas kernel here")


# speedup vs baseline: 4.5402x; 4.5402x over previous
import functools

import jax
import jax.numpy as jnp
from jax import lax
from jax.experimental import pallas as pl
from jax.experimental.pallas import tpu as pltpu
from jax.experimental.pallas import tpu_sc as plsc

_N = 10000
_E = 320000
_H = 128
_G = 256
_L = 64
_NW = 32
_EPW = _E // _NW
_C = 80
_NCH = _EPW // _C
_RPT = 624
_RCH = 104
_REM = _N - 16 * _RPT


def _sc_aggregate(h, edges_r):
    mesh = plsc.VectorSubcoreMesh(core_axis_name="c", subcore_axis_name="s")

    @functools.partial(
        pl.kernel,
        mesh=mesh,
        out_type=jax.ShapeDtypeStruct((2, _N, _H), jnp.float32),
        scratch_types=[
            pltpu.VMEM((_C,), jnp.int32),
            pltpu.VMEM((_C,), jnp.int32),
            pltpu.VMEM((_C, _H), jnp.float32),
            pltpu.VMEM((_RCH, _H), jnp.float32),
            pltpu.VMEM_SHARED((_N, _H), jnp.float32),
            pltpu.SemaphoreType.DMA,
        ],
    )
    def agg_kernel(h_hbm, e_hbm, out_hbm, src_v, dst_v, rows_v, zbuf, acc, sem):
        c = lax.axis_index("c")
        s = lax.axis_index("s")
        wid = s * 2 + c

        def zrow(i, carry):
            zbuf[i // 8, pl.ds((i % 8) * 16, 16)] = jnp.zeros((16,), jnp.float32)
            return carry

        lax.fori_loop(0, _RCH * 8, zrow, 0)
        for k in range(_RPT // _RCH):
            pltpu.sync_copy(zbuf, acc.at[pl.ds(s * _RPT + k * _RCH, _RCH)])

        @pl.when(s == 15)
        def _zero_rem():
            pltpu.sync_copy(zbuf.at[pl.ds(0, _REM)],
                            acc.at[pl.ds(16 * _RPT, _REM)])

        plsc.subcore_barrier()

        def chunk(j, carry):
            pltpu.sync_copy(e_hbm.at[0, wid, j], src_v)
            pltpu.sync_copy(e_hbm.at[1, wid, j], dst_v)
            pltpu.async_copy(h_hbm.at[src_v], rows_v, sem).wait()
            pltpu.sync_copy(rows_v, acc.at[dst_v], add=True)
            return carry

        lax.fori_loop(0, _NCH, chunk, 0)

        plsc.subcore_barrier()
        sl = pl.ds(s * _RPT, _RPT)
        pltpu.sync_copy(acc.at[sl], out_hbm.at[c, sl])

        @pl.when(s == 15)
        def _copy_rem():
            rem = pl.ds(16 * _RPT, _REM)
            pltpu.sync_copy(acc.at[rem], out_hbm.at[c, rem])

    return agg_kernel(h, edges_r)


def _tc_layer(h, agg, W1, b1, W2, b2, gamma, beta):
    def body(h_ref, a_ref, w1, bb1, w2, bb2, g, be, out):
        z = h_ref[...] + a_ref[0] + a_ref[1]
        y = jnp.maximum(
            jnp.dot(z, w1[...], preferred_element_type=jnp.float32) + bb1[...], 0.0)
        y = jnp.dot(y, w2[...], preferred_element_type=jnp.float32) + bb2[...]
        mean = jnp.mean(y, axis=0, keepdims=True)
        d = y - mean
        var = jnp.mean(d * d, axis=0, keepdims=True)
        out[...] = jnp.maximum(
            d * lax.rsqrt(var + 1e-5) * g[...] + be[...], 0.0)

    return pl.pallas_call(
        body,
        out_shape=jax.ShapeDtypeStruct((_N, _H), jnp.float32),
    )(h, agg, W1, b1.reshape(1, _H), W2, b2.reshape(1, _H),
      gamma.reshape(1, _H), beta.reshape(1, _H))


def _tc_final(h, agg, W1, b1, W2, b2, gamma, beta, batch2d, W_mu, b_mu, W_lv, b_lv):
    def body(h_ref, a_ref, w1, bb1, w2, bb2, g, be, bat, wmu, bmu, wlv, blv,
             mu_out, lv_out):
        z = h_ref[...] + a_ref[0] + a_ref[1]
        y = jnp.maximum(
            jnp.dot(z, w1[...], preferred_element_type=jnp.float32) + bb1[...], 0.0)
        y = jnp.dot(y, w2[...], preferred_element_type=jnp.float32) + bb2[...]
        mean = jnp.mean(y, axis=0, keepdims=True)
        d = y - mean
        var = jnp.mean(d * d, axis=0, keepdims=True)
        h3 = jnp.maximum(d * lax.rsqrt(var + 1e-5) * g[...] + be[...], 0.0)
        onehot = (bat[...] == lax.broadcasted_iota(jnp.int32, (_G, _N), 0)
                  ).astype(jnp.float32)
        cnt = jnp.sum(onehot, axis=1, keepdims=True)
        sums = jnp.dot(onehot, h3, preferred_element_type=jnp.float32)
        pooled = sums / jnp.maximum(cnt, 1.0)
        mu_out[...] = jnp.dot(pooled, wmu[...],
                              preferred_element_type=jnp.float32) + bmu[...]
        lv_out[...] = jnp.dot(pooled, wlv[...],
                              preferred_element_type=jnp.float32) + blv[...]

    return pl.pallas_call(
        body,
        out_shape=(jax.ShapeDtypeStruct((_G, _L), jnp.float32),
                   jax.ShapeDtypeStruct((_G, _L), jnp.float32)),
    )(h, agg, W1, b1.reshape(1, _H), W2, b2.reshape(1, _H),
      gamma.reshape(1, _H), beta.reshape(1, _H), batch2d,
      W_mu, b_mu.reshape(1, _L), W_lv, b_lv.reshape(1, _L))


def kernel(x, edge_index, batch, W1_0, b1_0, W2_0, b2_0, gamma_0, beta_0,
           W1_1, b1_1, W2_1, b2_1, gamma_1, beta_1,
           W1_2, b1_2, W2_2, b2_2, gamma_2, beta_2,
           W_mu, b_mu, W_lv, b_lv):
    edges_r = edge_index.reshape(2, _NW, _NCH, _C)
    batch2d = batch.reshape(1, _N)

    h = x
    layers = [
        (W1_0, b1_0, W2_0, b2_0, gamma_0, beta_0),
        (W1_1, b1_1, W2_1, b2_1, gamma_1, beta_1),
        (W1_2, b1_2, W2_2, b2_2, gamma_2, beta_2),
    ]
    for i, (W1, b1, W2, b2, g, be) in enumerate(layers):
        agg = _sc_aggregate(h, edges_r)
        if i < 2:
            h = _tc_layer(h, agg, W1, b1, W2, b2, g, be)
        else:
            return _tc_final(h, agg, W1, b1, W2, b2, g, be, batch2d,
                             W_mu, b_mu, W_lv, b_lv)


# speedup vs baseline: 8.5022x; 1.8726x over previous
import functools

import jax
import jax.numpy as jnp
from jax import lax
from jax.experimental import pallas as pl
from jax.experimental.pallas import tpu as pltpu
from jax.experimental.pallas import tpu_sc as plsc

_N = 10000
_E = 320000
_H = 128
_G = 256
_L = 64
_NW = 32
_EPW = _E // _NW
_C = 80
_NCH = _EPW // _C
_K = 25
_NBLK = _NCH // _K
_RPT = 624
_REM = _N - 16 * _RPT


def _sc_aggregate(h, edges_r):
    mesh = plsc.VectorSubcoreMesh(core_axis_name="c", subcore_axis_name="s")

    @functools.partial(
        pl.kernel,
        mesh=mesh,
        out_type=jax.ShapeDtypeStruct((2, _N, _H), jnp.float32),
        scratch_types=[
            pltpu.VMEM((2, _K, _C), jnp.int32),
            pltpu.VMEM((2, _K, _C), jnp.int32),
            pltpu.VMEM((2, _C, _H), jnp.float32),
            pltpu.VMEM_SHARED((_N, _H), jnp.float32),
            pltpu.SemaphoreType.DMA,
            pltpu.SemaphoreType.DMA,
            pltpu.SemaphoreType.DMA,
            pltpu.SemaphoreType.DMA,
        ],
    )
    def agg_kernel(h_hbm, e_hbm, out_hbm, src_i, dst_i, rows, acc,
                   sem_g, sem_s, sem_i, sem_z):
        c = lax.axis_index("c")
        s = lax.axis_index("s")
        wid = s * 2 + c

        def zrow(i, carry):
            rows[i // 640, (i // 8) % 80, pl.ds((i % 8) * 16, 16)] = (
                jnp.zeros((16,), jnp.float32))
            return carry

        lax.fori_loop(0, 2 * _C * 8, zrow, 0)
        base = s * _RPT
        for k in range(7):
            pltpu.async_copy(rows.at[0], acc.at[pl.ds(base + k * 80, 80)], sem_z)
        pltpu.async_copy(rows.at[0, pl.ds(0, 64)],
                         acc.at[pl.ds(base + 560, 64)], sem_z)

        @pl.when(s == 15)
        def _zero_rem():
            pltpu.async_copy(rows.at[0, pl.ds(0, _REM)],
                             acc.at[pl.ds(16 * _RPT, _REM)], sem_z)

        for k in range(7):
            pltpu.make_async_copy(rows.at[0], acc.at[pl.ds(base, 80)], sem_z).wait()
        pltpu.make_async_copy(rows.at[0, pl.ds(0, 64)],
                              acc.at[pl.ds(base, 64)], sem_z).wait()

        @pl.when(s == 15)
        def _zero_rem_wait():
            pltpu.make_async_copy(rows.at[0, pl.ds(0, _REM)],
                                  acc.at[pl.ds(0, _REM)], sem_z).wait()

        plsc.subcore_barrier()

        pltpu.sync_copy(e_hbm.at[0, wid, 0], src_i.at[0])
        pltpu.sync_copy(e_hbm.at[1, wid, 0], dst_i.at[0])
        pltpu.async_copy(e_hbm.at[0, wid, 1], src_i.at[1], sem_i)
        pltpu.async_copy(e_hbm.at[1, wid, 1], dst_i.at[1], sem_i)
        pltpu.async_copy(h_hbm.at[src_i.at[0, 0]], rows.at[0], sem_g)

        def step(j, carry):
            p = j % 2
            blk = j // _K
            pltpu.make_async_copy(h_hbm.at[src_i.at[blk % 2, j % _K]],
                                  rows.at[p], sem_g).wait()

            @pl.when(j >= 1)
            def _wait_prev_scatter():
                bj = (j - 1) // _K
                pltpu.make_async_copy(
                    rows.at[1 - p],
                    acc.at[dst_i.at[bj % 2, (j - 1) % _K]], sem_s).wait()

            @pl.when(jnp.logical_and(j % _K == 0,
                                     jnp.logical_and(j > 0, blk + 1 < _NBLK)))
            def _prefetch_idx():
                pltpu.async_copy(e_hbm.at[0, wid, blk + 1],
                                 src_i.at[(blk + 1) % 2], sem_i)
                pltpu.async_copy(e_hbm.at[1, wid, blk + 1],
                                 dst_i.at[(blk + 1) % 2], sem_i)

            @pl.when(j + 1 < _NCH)
            def _issue_next_gather():
                b1 = (j + 1) // _K

                @pl.when((j + 1) % _K == 0)
                def _wait_idx():
                    pltpu.make_async_copy(e_hbm.at[0, wid, b1],
                                          src_i.at[b1 % 2], sem_i).wait()
                    pltpu.make_async_copy(e_hbm.at[1, wid, b1],
                                          dst_i.at[b1 % 2], sem_i).wait()

                pltpu.async_copy(h_hbm.at[src_i.at[b1 % 2, (j + 1) % _K]],
                                 rows.at[1 - p], sem_g)

            pltpu.async_copy(rows.at[p], acc.at[dst_i.at[blk % 2, j % _K]],
                             sem_s, add=True)
            return carry

        lax.fori_loop(0, _NCH, step, 0)
        pltpu.make_async_copy(rows.at[(_NCH - 1) % 2],
                              acc.at[dst_i.at[(_NBLK - 1) % 2, _K - 1]],
                              sem_s).wait()

        plsc.subcore_barrier()
        sl = pl.ds(s * _RPT, _RPT)
        pltpu.sync_copy(acc.at[sl], out_hbm.at[c, sl])

        @pl.when(s == 15)
        def _copy_rem():
            rem = pl.ds(16 * _RPT, _REM)
            pltpu.sync_copy(acc.at[rem], out_hbm.at[c, rem])

    return agg_kernel(h, edges_r)


def _tc_layer(h, agg, W1, b1, W2, b2, gamma, beta):
    def body(h_ref, a_ref, w1, bb1, w2, bb2, g, be, out):
        z = h_ref[...] + a_ref[0] + a_ref[1]
        y = jnp.maximum(
            jnp.dot(z, w1[...], preferred_element_type=jnp.float32) + bb1[...], 0.0)
        y = jnp.dot(y, w2[...], preferred_element_type=jnp.float32) + bb2[...]
        mean = jnp.mean(y, axis=0, keepdims=True)
        d = y - mean
        var = jnp.mean(d * d, axis=0, keepdims=True)
        out[...] = jnp.maximum(
            d * lax.rsqrt(var + 1e-5) * g[...] + be[...], 0.0)

    return pl.pallas_call(
        body,
        out_shape=jax.ShapeDtypeStruct((_N, _H), jnp.float32),
    )(h, agg, W1, b1.reshape(1, _H), W2, b2.reshape(1, _H),
      gamma.reshape(1, _H), beta.reshape(1, _H))


def _tc_final(h, agg, W1, b1, W2, b2, gamma, beta, batch2d, W_mu, b_mu, W_lv, b_lv):
    def body(h_ref, a_ref, w1, bb1, w2, bb2, g, be, bat, wmu, bmu, wlv, blv,
             mu_out, lv_out):
        z = h_ref[...] + a_ref[0] + a_ref[1]
        y = jnp.maximum(
            jnp.dot(z, w1[...], preferred_element_type=jnp.float32) + bb1[...], 0.0)
        y = jnp.dot(y, w2[...], preferred_element_type=jnp.float32) + bb2[...]
        mean = jnp.mean(y, axis=0, keepdims=True)
        d = y - mean
        var = jnp.mean(d * d, axis=0, keepdims=True)
        h3 = jnp.maximum(d * lax.rsqrt(var + 1e-5) * g[...] + be[...], 0.0)
        onehot = (bat[...] == lax.broadcasted_iota(jnp.int32, (_G, _N), 0)
                  ).astype(jnp.float32)
        cnt = jnp.sum(onehot, axis=1, keepdims=True)
        sums = jnp.dot(onehot, h3, preferred_element_type=jnp.float32)
        pooled = sums / jnp.maximum(cnt, 1.0)
        mu_out[...] = jnp.dot(pooled, wmu[...],
                              preferred_element_type=jnp.float32) + bmu[...]
        lv_out[...] = jnp.dot(pooled, wlv[...],
                              preferred_element_type=jnp.float32) + blv[...]

    return pl.pallas_call(
        body,
        out_shape=(jax.ShapeDtypeStruct((_G, _L), jnp.float32),
                   jax.ShapeDtypeStruct((_G, _L), jnp.float32)),
    )(h, agg, W1, b1.reshape(1, _H), W2, b2.reshape(1, _H),
      gamma.reshape(1, _H), beta.reshape(1, _H), batch2d,
      W_mu, b_mu.reshape(1, _L), W_lv, b_lv.reshape(1, _L))


def kernel(x, edge_index, batch, W1_0, b1_0, W2_0, b2_0, gamma_0, beta_0,
           W1_1, b1_1, W2_1, b2_1, gamma_1, beta_1,
           W1_2, b1_2, W2_2, b2_2, gamma_2, beta_2,
           W_mu, b_mu, W_lv, b_lv):
    edges_r = edge_index.reshape(2, _NW, _NBLK, _K, _C)
    batch2d = batch.reshape(1, _N)

    h = x
    layers = [
        (W1_0, b1_0, W2_0, b2_0, gamma_0, beta_0),
        (W1_1, b1_1, W2_1, b2_1, gamma_1, beta_1),
        (W1_2, b1_2, W2_2, b2_2, gamma_2, beta_2),
    ]
    for i, (W1, b1, W2, b2, g, be) in enumerate(layers):
        agg = _sc_aggregate(h, edges_r)
        if i < 2:
            h = _tc_layer(h, agg, W1, b1, W2, b2, g, be)
        else:
            return _tc_final(h, agg, W1, b1, W2, b2, g, be, batch2d,
                             W_mu, b_mu, W_lv, b_lv)


# speedup vs baseline: 12.7818x; 1.5034x over previous
import functools

import jax
import jax.numpy as jnp
from jax import lax
from jax.experimental import pallas as pl
from jax.experimental.pallas import tpu as pltpu
from jax.experimental.pallas import tpu_sc as plsc

_N = 10000
_E = 320000
_H = 128
_G = 256
_L = 64
_NW = 32
_EPW = _E // _NW
_C = 80
_NCH = _EPW // _C
_K = 25
_NBLK = _NCH // _K
_RPT = 624
_REM = _N - 16 * _RPT


def _sc_aggregate(h, edges_r):
    mesh = plsc.VectorSubcoreMesh(core_axis_name="c", subcore_axis_name="s")

    @functools.partial(
        pl.kernel,
        mesh=mesh,
        out_type=jax.ShapeDtypeStruct((2, _N, _H), jnp.float32),
        scratch_types=[
            pltpu.VMEM((2, _K, _C), jnp.int32),
            pltpu.VMEM((2, _K, _C), jnp.int32),
            pltpu.VMEM((3, _C, _H), jnp.float32),
            pltpu.VMEM_SHARED((_N, _H), jnp.float32),
            pltpu.SemaphoreType.DMA,
            pltpu.SemaphoreType.DMA,
            pltpu.SemaphoreType.DMA,
            pltpu.SemaphoreType.DMA,
        ],
    )
    def agg_kernel(h_hbm, e_hbm, out_hbm, src_i, dst_i, rows, acc,
                   sem_g, sem_s, sem_i, sem_z):
        c = lax.axis_index("c")
        s = lax.axis_index("s")
        wid = s * 2 + c

        def zrow(i, carry):
            rows[0, i // 8, pl.ds((i % 8) * 16, 16)] = (
                jnp.zeros((16,), jnp.float32))
            return carry

        lax.fori_loop(0, _C * 8, zrow, 0)
        base = s * _RPT
        for k in range(7):
            pltpu.async_copy(rows.at[0], acc.at[pl.ds(base + k * 80, 80)], sem_z)
        pltpu.async_copy(rows.at[0, pl.ds(0, 64)],
                         acc.at[pl.ds(base + 560, 64)], sem_z)

        @pl.when(s == 15)
        def _zero_rem():
            pltpu.async_copy(rows.at[0, pl.ds(0, _REM)],
                             acc.at[pl.ds(16 * _RPT, _REM)], sem_z)

        for k in range(7):
            pltpu.make_async_copy(rows.at[0], acc.at[pl.ds(base, 80)], sem_z).wait()
        pltpu.make_async_copy(rows.at[0, pl.ds(0, 64)],
                              acc.at[pl.ds(base, 64)], sem_z).wait()

        @pl.when(s == 15)
        def _zero_rem_wait():
            pltpu.make_async_copy(rows.at[0, pl.ds(0, _REM)],
                                  acc.at[pl.ds(0, _REM)], sem_z).wait()

        plsc.subcore_barrier()

        pltpu.sync_copy(e_hbm.at[0, wid, 0], src_i.at[0])
        pltpu.sync_copy(e_hbm.at[1, wid, 0], dst_i.at[0])
        pltpu.async_copy(e_hbm.at[0, wid, 1], src_i.at[1], sem_i)
        pltpu.async_copy(e_hbm.at[1, wid, 1], dst_i.at[1], sem_i)
        pltpu.async_copy(h_hbm.at[src_i.at[0, 0]], rows.at[0], sem_g)
        pltpu.async_copy(h_hbm.at[src_i.at[0, 1]], rows.at[1], sem_g)

        def step(j, carry):
            p = j % 3
            blk = j // _K
            pltpu.make_async_copy(h_hbm.at[src_i.at[blk % 2, j % _K]],
                                  rows.at[p], sem_g).wait()

            @pl.when(j >= 1)
            def _wait_prev_scatter():
                bj = (j - 1) // _K
                pltpu.make_async_copy(
                    rows.at[(j - 1) % 3],
                    acc.at[dst_i.at[bj % 2, (j - 1) % _K]], sem_s).wait()

            @pl.when(jnp.logical_and(j % _K == 2,
                                     jnp.logical_and(blk >= 1,
                                                     blk + 1 < _NBLK)))
            def _prefetch_idx():
                pltpu.async_copy(e_hbm.at[0, wid, blk + 1],
                                 src_i.at[(blk + 1) % 2], sem_i)
                pltpu.async_copy(e_hbm.at[1, wid, blk + 1],
                                 dst_i.at[(blk + 1) % 2], sem_i)

            @pl.when(j + 2 < _NCH)
            def _issue_next_gather():
                b2 = (j + 2) // _K

                @pl.when((j + 2) % _K == 0)
                def _wait_idx():
                    pltpu.make_async_copy(e_hbm.at[0, wid, b2],
                                          src_i.at[b2 % 2], sem_i).wait()
                    pltpu.make_async_copy(e_hbm.at[1, wid, b2],
                                          dst_i.at[b2 % 2], sem_i).wait()

                pltpu.async_copy(h_hbm.at[src_i.at[b2 % 2, (j + 2) % _K]],
                                 rows.at[(j + 2) % 3], sem_g)

            pltpu.async_copy(rows.at[p], acc.at[dst_i.at[blk % 2, j % _K]],
                             sem_s, add=True)
            return carry

        lax.fori_loop(0, _NCH, step, 0)
        pltpu.make_async_copy(
            rows.at[(_NCH - 1) % 3],
            acc.at[dst_i.at[((_NCH - 1) // _K) % 2, (_NCH - 1) % _K]],
            sem_s).wait()

        plsc.subcore_barrier()
        sl = pl.ds(s * _RPT, _RPT)
        pltpu.sync_copy(acc.at[sl], out_hbm.at[c, sl])

        @pl.when(s == 15)
        def _copy_rem():
            rem = pl.ds(16 * _RPT, _REM)
            pltpu.sync_copy(acc.at[rem], out_hbm.at[c, rem])

    return agg_kernel(h, edges_r)


def _tc_layer(h, agg, W1, b1, W2, b2, gamma, beta):
    def body(h_ref, a_ref, w1, bb1, w2, bb2, g, be, out):
        z = h_ref[...] + a_ref[0] + a_ref[1]
        y = jnp.maximum(
            jnp.dot(z, w1[...], preferred_element_type=jnp.float32) + bb1[...], 0.0)
        y = jnp.dot(y, w2[...], preferred_element_type=jnp.float32) + bb2[...]
        mean = jnp.mean(y, axis=0, keepdims=True)
        d = y - mean
        var = jnp.mean(d * d, axis=0, keepdims=True)
        out[...] = jnp.maximum(
            d * lax.rsqrt(var + 1e-5) * g[...] + be[...], 0.0)

    return pl.pallas_call(
        body,
        out_shape=jax.ShapeDtypeStruct((_N, _H), jnp.float32),
    )(h, agg, W1, b1.reshape(1, _H), W2, b2.reshape(1, _H),
      gamma.reshape(1, _H), beta.reshape(1, _H))


def _tc_final(h, agg, W1, b1, W2, b2, gamma, beta, batch2d, W_mu, b_mu, W_lv, b_lv):
    def body(h_ref, a_ref, w1, bb1, w2, bb2, g, be, bat, wmu, bmu, wlv, blv,
             mu_out, lv_out):
        z = h_ref[...] + a_ref[0] + a_ref[1]
        y = jnp.maximum(
            jnp.dot(z, w1[...], preferred_element_type=jnp.float32) + bb1[...], 0.0)
        y = jnp.dot(y, w2[...], preferred_element_type=jnp.float32) + bb2[...]
        mean = jnp.mean(y, axis=0, keepdims=True)
        d = y - mean
        var = jnp.mean(d * d, axis=0, keepdims=True)
        h3 = jnp.maximum(d * lax.rsqrt(var + 1e-5) * g[...] + be[...], 0.0)
        onehot = (bat[...] == lax.broadcasted_iota(jnp.int32, (_G, _N), 0)
                  ).astype(jnp.float32)
        cnt = jnp.sum(onehot, axis=1, keepdims=True)
        sums = jnp.dot(onehot, h3, preferred_element_type=jnp.float32)
        pooled = sums / jnp.maximum(cnt, 1.0)
        mu_out[...] = jnp.dot(pooled, wmu[...],
                              preferred_element_type=jnp.float32) + bmu[...]
        lv_out[...] = jnp.dot(pooled, wlv[...],
                              preferred_element_type=jnp.float32) + blv[...]

    return pl.pallas_call(
        body,
        out_shape=(jax.ShapeDtypeStruct((_G, _L), jnp.float32),
                   jax.ShapeDtypeStruct((_G, _L), jnp.float32)),
    )(h, agg, W1, b1.reshape(1, _H), W2, b2.reshape(1, _H),
      gamma.reshape(1, _H), beta.reshape(1, _H), batch2d,
      W_mu, b_mu.reshape(1, _L), W_lv, b_lv.reshape(1, _L))


def kernel(x, edge_index, batch, W1_0, b1_0, W2_0, b2_0, gamma_0, beta_0,
           W1_1, b1_1, W2_1, b2_1, gamma_1, beta_1,
           W1_2, b1_2, W2_2, b2_2, gamma_2, beta_2,
           W_mu, b_mu, W_lv, b_lv):
    edges_r = edge_index.reshape(2, _NW, _NBLK, _K, _C)
    batch2d = batch.reshape(1, _N)

    h = x
    layers = [
        (W1_0, b1_0, W2_0, b2_0, gamma_0, beta_0),
        (W1_1, b1_1, W2_1, b2_1, gamma_1, beta_1),
        (W1_2, b1_2, W2_2, b2_2, gamma_2, beta_2),
    ]
    for i, (W1, b1, W2, b2, g, be) in enumerate(layers):
        agg = _sc_aggregate(h, edges_r)
        if i < 2:
            h = _tc_layer(h, agg, W1, b1, W2, b2, g, be)
        else:
            return _tc_final(h, agg, W1, b1, W2, b2, g, be, batch2d,
                             W_mu, b_mu, W_lv, b_lv)


# speedup vs baseline: 13.2138x; 1.0338x over previous
import functools

import jax
import jax.numpy as jnp
from jax import lax
from jax.experimental import pallas as pl
from jax.experimental.pallas import tpu as pltpu
from jax.experimental.pallas import tpu_sc as plsc

_N = 10000
_E = 320000
_H = 128
_G = 256
_L = 64
_NW = 32
_EPW = _E // _NW
_C = 50
_NCH = _EPW // _C
_K = 10
_NBLK = _NCH // _K
_NB = 5
_RPT = 624
_REM = _N - 16 * _RPT


def _sc_aggregate(h, edges_r):
    mesh = plsc.VectorSubcoreMesh(core_axis_name="c", subcore_axis_name="s")

    @functools.partial(
        pl.kernel,
        mesh=mesh,
        out_type=jax.ShapeDtypeStruct((2, _N, _H), jnp.float32),
        scratch_types=[
            pltpu.VMEM((2, _K, _C), jnp.int32),
            pltpu.VMEM((2, _K, _C), jnp.int32),
            pltpu.VMEM((_NB, _C, _H), jnp.float32),
            pltpu.VMEM_SHARED((_N, _H), jnp.float32),
            pltpu.SemaphoreType.DMA,
            pltpu.SemaphoreType.DMA,
            pltpu.SemaphoreType.DMA,
            pltpu.SemaphoreType.DMA,
        ],
    )
    def agg_kernel(h_hbm, e_hbm, out_hbm, src_i, dst_i, rows, acc,
                   sem_g, sem_s, sem_i, sem_z):
        c = lax.axis_index("c")
        s = lax.axis_index("s")
        wid = s * 2 + c

        def zrow(i, carry):
            rows[0, i // 8, pl.ds((i % 8) * 16, 16)] = (
                jnp.zeros((16,), jnp.float32))
            return carry

        lax.fori_loop(0, _C * 8, zrow, 0)
        base = s * _RPT
        zq, zr = _RPT // _C, _RPT % _C
        for k in range(zq):
            pltpu.async_copy(rows.at[0], acc.at[pl.ds(base + k * _C, _C)], sem_z)
        if zr:
            pltpu.async_copy(rows.at[0, pl.ds(0, zr)],
                             acc.at[pl.ds(base + zq * _C, zr)], sem_z)

        @pl.when(s == 15)
        def _zero_rem():
            pltpu.async_copy(rows.at[0, pl.ds(0, _REM)],
                             acc.at[pl.ds(16 * _RPT, _REM)], sem_z)

        for k in range(zq):
            pltpu.make_async_copy(rows.at[0], acc.at[pl.ds(base, _C)], sem_z).wait()
        if zr:
            pltpu.make_async_copy(rows.at[0, pl.ds(0, zr)],
                                  acc.at[pl.ds(base, zr)], sem_z).wait()

        @pl.when(s == 15)
        def _zero_rem_wait():
            pltpu.make_async_copy(rows.at[0, pl.ds(0, _REM)],
                                  acc.at[pl.ds(0, _REM)], sem_z).wait()

        plsc.subcore_barrier()

        pltpu.sync_copy(e_hbm.at[0, wid, 0], src_i.at[0])
        pltpu.sync_copy(e_hbm.at[1, wid, 0], dst_i.at[0])
        pltpu.async_copy(e_hbm.at[0, wid, 1], src_i.at[1], sem_i)
        pltpu.async_copy(e_hbm.at[1, wid, 1], dst_i.at[1], sem_i)
        for q in range(_NB - 1):
            pltpu.async_copy(h_hbm.at[src_i.at[0, q]], rows.at[q], sem_g)

        def step(j, carry):
            p = j % _NB
            blk = j // _K
            pltpu.make_async_copy(h_hbm.at[src_i.at[blk % 2, j % _K]],
                                  rows.at[p], sem_g).wait()

            @pl.when(j >= 1)
            def _wait_prev_scatter():
                bj = (j - 1) // _K
                pltpu.make_async_copy(
                    rows.at[(j - 1) % _NB],
                    acc.at[dst_i.at[bj % 2, (j - 1) % _K]], sem_s).wait()

            @pl.when(jnp.logical_and(j % _K == _NB - 1,
                                     jnp.logical_and(blk >= 1,
                                                     blk + 1 < _NBLK)))
            def _prefetch_idx():
                pltpu.async_copy(e_hbm.at[0, wid, blk + 1],
                                 src_i.at[(blk + 1) % 2], sem_i)
                pltpu.async_copy(e_hbm.at[1, wid, blk + 1],
                                 dst_i.at[(blk + 1) % 2], sem_i)

            @pl.when(j + _NB - 1 < _NCH)
            def _issue_next_gather():
                jn = j + _NB - 1
                b2 = jn // _K

                @pl.when(jn % _K == 0)
                def _wait_idx():
                    pltpu.make_async_copy(e_hbm.at[0, wid, b2],
                                          src_i.at[b2 % 2], sem_i).wait()
                    pltpu.make_async_copy(e_hbm.at[1, wid, b2],
                                          dst_i.at[b2 % 2], sem_i).wait()

                pltpu.async_copy(h_hbm.at[src_i.at[b2 % 2, jn % _K]],
                                 rows.at[jn % _NB], sem_g)

            pltpu.async_copy(rows.at[p], acc.at[dst_i.at[blk % 2, j % _K]],
                             sem_s, add=True)
            return carry

        lax.fori_loop(0, _NCH, step, 0)
        pltpu.make_async_copy(
            rows.at[(_NCH - 1) % _NB],
            acc.at[dst_i.at[((_NCH - 1) // _K) % 2, (_NCH - 1) % _K]],
            sem_s).wait()

        plsc.subcore_barrier()
        sl = pl.ds(s * _RPT, _RPT)
        pltpu.sync_copy(acc.at[sl], out_hbm.at[c, sl])

        @pl.when(s == 15)
        def _copy_rem():
            rem = pl.ds(16 * _RPT, _REM)
            pltpu.sync_copy(acc.at[rem], out_hbm.at[c, rem])

    return agg_kernel(h, edges_r)


def _tc_layer(h, agg, W1, b1, W2, b2, gamma, beta):
    def body(h_ref, a_ref, w1, bb1, w2, bb2, g, be, out):
        z = h_ref[...] + a_ref[0] + a_ref[1]
        y = jnp.maximum(
            jnp.dot(z, w1[...], preferred_element_type=jnp.float32) + bb1[...], 0.0)
        y = jnp.dot(y, w2[...], preferred_element_type=jnp.float32) + bb2[...]
        mean = jnp.mean(y, axis=0, keepdims=True)
        d = y - mean
        var = jnp.mean(d * d, axis=0, keepdims=True)
        out[...] = jnp.maximum(
            d * lax.rsqrt(var + 1e-5) * g[...] + be[...], 0.0)

    return pl.pallas_call(
        body,
        out_shape=jax.ShapeDtypeStruct((_N, _H), jnp.float32),
    )(h, agg, W1, b1.reshape(1, _H), W2, b2.reshape(1, _H),
      gamma.reshape(1, _H), beta.reshape(1, _H))


def _tc_final(h, agg, W1, b1, W2, b2, gamma, beta, batch2d, W_mu, b_mu, W_lv, b_lv):
    def body(h_ref, a_ref, w1, bb1, w2, bb2, g, be, bat, wmu, bmu, wlv, blv,
             mu_out, lv_out):
        z = h_ref[...] + a_ref[0] + a_ref[1]
        y = jnp.maximum(
            jnp.dot(z, w1[...], preferred_element_type=jnp.float32) + bb1[...], 0.0)
        y = jnp.dot(y, w2[...], preferred_element_type=jnp.float32) + bb2[...]
        mean = jnp.mean(y, axis=0, keepdims=True)
        d = y - mean
        var = jnp.mean(d * d, axis=0, keepdims=True)
        h3 = jnp.maximum(d * lax.rsqrt(var + 1e-5) * g[...] + be[...], 0.0)
        onehot = (bat[...] == lax.broadcasted_iota(jnp.int32, (_G, _N), 0)
                  ).astype(jnp.float32)
        cnt = jnp.sum(onehot, axis=1, keepdims=True)
        sums = jnp.dot(onehot, h3, preferred_element_type=jnp.float32)
        pooled = sums / jnp.maximum(cnt, 1.0)
        mu_out[...] = jnp.dot(pooled, wmu[...],
                              preferred_element_type=jnp.float32) + bmu[...]
        lv_out[...] = jnp.dot(pooled, wlv[...],
                              preferred_element_type=jnp.float32) + blv[...]

    return pl.pallas_call(
        body,
        out_shape=(jax.ShapeDtypeStruct((_G, _L), jnp.float32),
                   jax.ShapeDtypeStruct((_G, _L), jnp.float32)),
    )(h, agg, W1, b1.reshape(1, _H), W2, b2.reshape(1, _H),
      gamma.reshape(1, _H), beta.reshape(1, _H), batch2d,
      W_mu, b_mu.reshape(1, _L), W_lv, b_lv.reshape(1, _L))


def kernel(x, edge_index, batch, W1_0, b1_0, W2_0, b2_0, gamma_0, beta_0,
           W1_1, b1_1, W2_1, b2_1, gamma_1, beta_1,
           W1_2, b1_2, W2_2, b2_2, gamma_2, beta_2,
           W_mu, b_mu, W_lv, b_lv):
    edges_r = edge_index.reshape(2, _NW, _NBLK, _K, _C)
    batch2d = batch.reshape(1, _N)

    h = x
    layers = [
        (W1_0, b1_0, W2_0, b2_0, gamma_0, beta_0),
        (W1_1, b1_1, W2_1, b2_1, gamma_1, beta_1),
        (W1_2, b1_2, W2_2, b2_2, gamma_2, beta_2),
    ]
    for i, (W1, b1, W2, b2, g, be) in enumerate(layers):
        agg = _sc_aggregate(h, edges_r)
        if i < 2:
            h = _tc_layer(h, agg, W1, b1, W2, b2, g, be)
        else:
            return _tc_final(h, agg, W1, b1, W2, b2, g, be, batch2d,
                             W_mu, b_mu, W_lv, b_lv)


# speedup vs baseline: 13.2340x; 1.0015x over previous
import functools

import jax
import jax.numpy as jnp
from jax import lax
from jax.experimental import pallas as pl
from jax.experimental.pallas import tpu as pltpu
from jax.experimental.pallas import tpu_sc as plsc

_N = 10000
_E = 320000
_H = 128
_G = 256
_L = 64
_NW = 32
_EPW = _E // _NW
_C = 50
_NCH = _EPW // _C
_K = 10
_NBLK = _NCH // _K
_NB = 5
_RPT = 624
_REM = _N - 16 * _RPT


def _sc_aggregate(h, edges_r):
    mesh = plsc.VectorSubcoreMesh(core_axis_name="c", subcore_axis_name="s")

    @functools.partial(
        pl.kernel,
        mesh=mesh,
        out_type=jax.ShapeDtypeStruct((2, _N, _H), jnp.float32),
        scratch_types=[
            pltpu.VMEM((2, _K, _C), jnp.int32),
            pltpu.VMEM((2, _K, _C), jnp.int32),
            pltpu.VMEM((_NB, _C, _H), jnp.float32),
            pltpu.VMEM_SHARED((_N, _H), jnp.float32),
            pltpu.SemaphoreType.DMA,
            pltpu.SemaphoreType.DMA,
            pltpu.SemaphoreType.DMA,
            pltpu.SemaphoreType.DMA,
        ],
    )
    def agg_kernel(h_hbm, e_hbm, out_hbm, src_i, dst_i, rows, acc,
                   sem_g, sem_s, sem_i, sem_z):
        c = lax.axis_index("c")
        s = lax.axis_index("s")
        wid = s * 2 + c

        def zrow(i, carry):
            rows[0, i // 8, pl.ds((i % 8) * 16, 16)] = (
                jnp.zeros((16,), jnp.float32))
            return carry

        lax.fori_loop(0, _C * 8, zrow, 0)
        base = s * _RPT
        zq, zr = _RPT // _C, _RPT % _C
        for k in range(zq):
            pltpu.async_copy(rows.at[0], acc.at[pl.ds(base + k * _C, _C)], sem_z)
        if zr:
            pltpu.async_copy(rows.at[0, pl.ds(0, zr)],
                             acc.at[pl.ds(base + zq * _C, zr)], sem_z)

        @pl.when(s == 15)
        def _zero_rem():
            pltpu.async_copy(rows.at[0, pl.ds(0, _REM)],
                             acc.at[pl.ds(16 * _RPT, _REM)], sem_z)

        for k in range(zq):
            pltpu.make_async_copy(rows.at[0], acc.at[pl.ds(base, _C)], sem_z).wait()
        if zr:
            pltpu.make_async_copy(rows.at[0, pl.ds(0, zr)],
                                  acc.at[pl.ds(base, zr)], sem_z).wait()

        @pl.when(s == 15)
        def _zero_rem_wait():
            pltpu.make_async_copy(rows.at[0, pl.ds(0, _REM)],
                                  acc.at[pl.ds(0, _REM)], sem_z).wait()

        pltpu.sync_copy(e_hbm.at[0, wid, 0], src_i.at[0])
        pltpu.sync_copy(e_hbm.at[1, wid, 0], dst_i.at[0])
        pltpu.async_copy(e_hbm.at[0, wid, 1], src_i.at[1], sem_i)
        pltpu.async_copy(e_hbm.at[1, wid, 1], dst_i.at[1], sem_i)
        for q in range(_NB - 1):
            pltpu.async_copy(h_hbm.at[src_i.at[0, q]], rows.at[q], sem_g)

        plsc.subcore_barrier()

        def step(j, carry):
            p = j % _NB
            blk = j // _K
            pltpu.make_async_copy(h_hbm.at[src_i.at[blk % 2, j % _K]],
                                  rows.at[p], sem_g).wait()

            @pl.when(j >= 1)
            def _wait_prev_scatter():
                bj = (j - 1) // _K
                pltpu.make_async_copy(
                    rows.at[(j - 1) % _NB],
                    acc.at[dst_i.at[bj % 2, (j - 1) % _K]], sem_s).wait()

            @pl.when(jnp.logical_and(j % _K == _NB - 1,
                                     jnp.logical_and(blk >= 1,
                                                     blk + 1 < _NBLK)))
            def _prefetch_idx():
                pltpu.async_copy(e_hbm.at[0, wid, blk + 1],
                                 src_i.at[(blk + 1) % 2], sem_i)
                pltpu.async_copy(e_hbm.at[1, wid, blk + 1],
                                 dst_i.at[(blk + 1) % 2], sem_i)

            @pl.when(j + _NB - 1 < _NCH)
            def _issue_next_gather():
                jn = j + _NB - 1
                b2 = jn // _K

                @pl.when(jn % _K == 0)
                def _wait_idx():
                    pltpu.make_async_copy(e_hbm.at[0, wid, b2],
                                          src_i.at[b2 % 2], sem_i).wait()
                    pltpu.make_async_copy(e_hbm.at[1, wid, b2],
                                          dst_i.at[b2 % 2], sem_i).wait()

                pltpu.async_copy(h_hbm.at[src_i.at[b2 % 2, jn % _K]],
                                 rows.at[jn % _NB], sem_g)

            pltpu.async_copy(rows.at[p], acc.at[dst_i.at[blk % 2, j % _K]],
                             sem_s, add=True)
            return carry

        lax.fori_loop(0, _NCH, step, 0)
        pltpu.make_async_copy(
            rows.at[(_NCH - 1) % _NB],
            acc.at[dst_i.at[((_NCH - 1) // _K) % 2, (_NCH - 1) % _K]],
            sem_s).wait()

        plsc.subcore_barrier()
        sl = pl.ds(s * _RPT, _RPT)
        pltpu.sync_copy(acc.at[sl], out_hbm.at[c, sl])

        @pl.when(s == 15)
        def _copy_rem():
            rem = pl.ds(16 * _RPT, _REM)
            pltpu.sync_copy(acc.at[rem], out_hbm.at[c, rem])

    return agg_kernel(h, edges_r)


def _tc_layer(h, agg, W1, b1, W2, b2, gamma, beta):
    def body(h_ref, a_ref, w1, bb1, w2, bb2, g, be, out):
        z = h_ref[...] + a_ref[0] + a_ref[1]
        y = jnp.maximum(
            jnp.dot(z, w1[...], preferred_element_type=jnp.float32) + bb1[...], 0.0)
        y = jnp.dot(y, w2[...], preferred_element_type=jnp.float32) + bb2[...]
        mean = jnp.mean(y, axis=0, keepdims=True)
        d = y - mean
        var = jnp.mean(d * d, axis=0, keepdims=True)
        out[...] = jnp.maximum(
            d * lax.rsqrt(var + 1e-5) * g[...] + be[...], 0.0)

    return pl.pallas_call(
        body,
        out_shape=jax.ShapeDtypeStruct((_N, _H), jnp.float32),
    )(h, agg, W1, b1.reshape(1, _H), W2, b2.reshape(1, _H),
      gamma.reshape(1, _H), beta.reshape(1, _H))


def _tc_final(h, agg, W1, b1, W2, b2, gamma, beta, batch2d, W_mu, b_mu, W_lv, b_lv):
    def body(h_ref, a_ref, w1, bb1, w2, bb2, g, be, bat, wmu, bmu, wlv, blv,
             mu_out, lv_out):
        z = h_ref[...] + a_ref[0] + a_ref[1]
        y = jnp.maximum(
            jnp.dot(z, w1[...], preferred_element_type=jnp.float32) + bb1[...], 0.0)
        y = jnp.dot(y, w2[...], preferred_element_type=jnp.float32) + bb2[...]
        mean = jnp.mean(y, axis=0, keepdims=True)
        d = y - mean
        var = jnp.mean(d * d, axis=0, keepdims=True)
        h3 = jnp.maximum(d * lax.rsqrt(var + 1e-5) * g[...] + be[...], 0.0)
        onehot = (bat[...] == lax.broadcasted_iota(jnp.int32, (_G, _N), 0)
                  ).astype(jnp.float32)
        cnt = jnp.sum(onehot, axis=1, keepdims=True)
        sums = jnp.dot(onehot, h3, preferred_element_type=jnp.float32)
        pooled = sums / jnp.maximum(cnt, 1.0)
        mu_out[...] = jnp.dot(pooled, wmu[...],
                              preferred_element_type=jnp.float32) + bmu[...]
        lv_out[...] = jnp.dot(pooled, wlv[...],
                              preferred_element_type=jnp.float32) + blv[...]

    return pl.pallas_call(
        body,
        out_shape=(jax.ShapeDtypeStruct((_G, _L), jnp.float32),
                   jax.ShapeDtypeStruct((_G, _L), jnp.float32)),
    )(h, agg, W1, b1.reshape(1, _H), W2, b2.reshape(1, _H),
      gamma.reshape(1, _H), beta.reshape(1, _H), batch2d,
      W_mu, b_mu.reshape(1, _L), W_lv, b_lv.reshape(1, _L))


def kernel(x, edge_index, batch, W1_0, b1_0, W2_0, b2_0, gamma_0, beta_0,
           W1_1, b1_1, W2_1, b2_1, gamma_1, beta_1,
           W1_2, b1_2, W2_2, b2_2, gamma_2, beta_2,
           W_mu, b_mu, W_lv, b_lv):
    edges_r = edge_index.reshape(2, _NW, _NBLK, _K, _C)
    batch2d = batch.reshape(1, _N)

    h = x
    layers = [
        (W1_0, b1_0, W2_0, b2_0, gamma_0, beta_0),
        (W1_1, b1_1, W2_1, b2_1, gamma_1, beta_1),
        (W1_2, b1_2, W2_2, b2_2, gamma_2, beta_2),
    ]
    for i, (W1, b1, W2, b2, g, be) in enumerate(layers):
        agg = _sc_aggregate(h, edges_r)
        if i < 2:
            h = _tc_layer(h, agg, W1, b1, W2, b2, g, be)
        else:
            return _tc_final(h, agg, W1, b1, W2, b2, g, be, batch2d,
                             W_mu, b_mu, W_lv, b_lv)
